# baseline (speedup 1.0000x reference)
import functools

import jax
import jax.numpy as jnp
from jax import lax
from jax.experimental import pallas as pl
from jax.experimental.pallas import tpu as pltpu
from jax.experimental.pallas import tpu_sc as plsc

NUM_EMB = 1000000
DIM = 32
BATCH = 16384
FIELDS = 26
B = BATCH * FIELDS

NC = 2
NS = 16
NW = NC * NS
TB = BATCH // 128
NBLK = FIELDS * TB
BLK_PER_W = NBLK // NW
NPAIR = BLK_PER_W // 2
IDX_PER_W = BLK_PER_W * 128

_mesh = plsc.VectorSubcoreMesh(core_axis_name="c", subcore_axis_name="s")

NPANEL = 7812
SLAB = 128
NSLAB = NPANEL * 128 // SLAB
SLAB_PER_W = 246


@functools.partial(
    pl.kernel,
    mesh=_mesh,
    out_type=jax.ShapeDtypeStruct((NUM_EMB // 4, 128), jnp.float32),
    compiler_params=pltpu.CompilerParams(
        use_tc_tiling_on_sc=True, needs_layout_passes=False),
    scratch_types=[
        pltpu.VMEM((2, DIM, SLAB + 1), jnp.float32),
        pltpu.VMEM((2, SLAB // 4, 128), jnp.float32),
        pltpu.SemaphoreType.DMA,
        pltpu.SemaphoreType.DMA,
        pltpu.SemaphoreType.DMA,
        pltpu.SemaphoreType.DMA,
    ],
)
def _format(tw_hbm, tail4_hbm, out_hbm, src_v, dst_v, rsem0, rsem1, wsem0,
            wsem1):
    wid = lax.axis_index("s") * NC + lax.axis_index("c")
    rsems = (rsem0, rsem1)
    wsems = (wsem0, wsem1)

    def slab_of(t):
        return t * NW + wid

    def fire_read(t, b):
        pltpu.async_copy(
            tw_hbm.at[:, pl.ds(slab_of(t) * SLAB, SLAB)],
            src_v.at[b, :, pl.ds(0, SLAB)], rsems[b])

    def drain_read(t, b):
        pltpu.make_async_copy(
            tw_hbm.at[:, pl.ds(slab_of(t) * SLAB, SLAB)],
            src_v.at[b, :, pl.ds(0, SLAB)], rsems[b]).wait()

    def transpose(b):
        rids = []
        cids = []
        for h in range(8):
            pvec = lax.iota(jnp.int32, 16) + h * 16
            rids.append(lax.bitwise_and(pvec, 31))
            cids.append(lax.shift_right_logical(pvec, 5))

        @plsc.parallel_loop(0, SLAB // 4, step=1, unroll=4)
        def _(s):
            for h in range(8):
                val = plsc.load_gather(
                    src_v.at[b], [rids[h], cids[h] + s * 4])
                dst_v[b, s, pl.ds(h * 16, 16)] = val

    def fire_write(t, b):
        pltpu.async_copy(
            dst_v.at[b],
            out_hbm.at[pl.ds(slab_of(t) * (SLAB // 4), SLAB // 4)], wsems[b])

    def wait_write(b):
        pltpu.make_async_copy(
            dst_v.at[b], out_hbm.at[pl.ds(0, SLAB // 4)], wsems[b]).wait()

    @pl.when(slab_of(0) < NSLAB)
    def _():
        fire_read(0, 0)

    def half(t, b, m):
        @pl.when(slab_of(t + 1) < NSLAB)
        def _():
            fire_read(t + 1, 1 - b)

        @pl.when(slab_of(t) < NSLAB)
        def _():
            drain_read(t, b)

            @pl.when(m >= 1)
            def _():
                wait_write(b)

            transpose(b)
            fire_write(t, b)

    def body(m, carry):
        half(m * 2, 0, m)
        half(m * 2 + 1, 1, m)
        return carry

    lax.fori_loop(0, SLAB_PER_W // 2, body, 0)
    wait_write(0)
    wait_write(1)

    @pl.when(wid == 0)
    def _():
        pltpu.sync_copy(tail4_hbm, dst_v.at[0, pl.ds(0, 16)])
        pltpu.sync_copy(
            dst_v.at[0, pl.ds(0, 16)], out_hbm.at[pl.ds(NPANEL * 32, 16)])


@functools.partial(
    pl.kernel,
    mesh=_mesh,
    out_type=jax.ShapeDtypeStruct((FIELDS, DIM, BATCH), jnp.float32),
    compiler_params=pltpu.CompilerParams(
        use_tc_tiling_on_sc=True, needs_layout_passes=False),
    scratch_types=[
        pltpu.VMEM((IDX_PER_W,), jnp.int32),
        pltpu.VMEM((2, 128), jnp.int32),
        pltpu.VMEM((2, 128), jnp.int32),
        pltpu.VMEM((2, 128, 128), jnp.float32),
        pltpu.VMEM((2, DIM, 128), jnp.float32),
        pltpu.SemaphoreType.DMA,
        pltpu.SemaphoreType.DMA,
        pltpu.SemaphoreType.DMA,
        pltpu.SemaphoreType.DMA,
    ],
)
def _embed(idx_hbm, table_hbm, out_hbm, idx_v, sid_v, cb_v, rows_v, ot_v,
           gsem0, gsem1, osem0, osem1):
    wid = lax.axis_index("s") * NC + lax.axis_index("c")
    base_blk = wid * BLK_PER_W
    pltpu.sync_copy(idx_hbm.at[pl.ds(wid * IDX_PER_W, IDX_PER_W)], idx_v)

    gsems = (gsem0, gsem1)
    osems = (osem0, osem1)

    def prep(k, b):
        for g in range(8):
            v = idx_v[pl.ds(k * 128 + g * 16, 16)]
            sid_v[b, pl.ds(g * 16, 16)] = lax.shift_right_logical(v, 2)
            cb_v[b, pl.ds(g * 16, 16)] = lax.shift_left(
                lax.bitwise_and(v, 3), 5)

    def fire(b):
        pltpu.async_copy(table_hbm.at[sid_v.at[b]], rows_v.at[b], gsems[b])

    def drain(b):
        pltpu.make_async_copy(
            table_hbm.at[sid_v.at[b]], rows_v.at[b], gsems[b]).wait()

    def extract(b):
        rids = [lax.iota(jnp.int32, 16) + g * 16 for g in range(8)]
        cbs = [cb_v[b, pl.ds(g * 16, 16)] for g in range(8)]

        @plsc.parallel_loop(0, DIM, step=1, unroll=4)
        def _(d):
            for g in range(8):
                val = plsc.load_gather(rows_v.at[b], [rids[g], cbs[g] + d])
                ot_v[b, d, pl.ds(g * 16, 16)] = val

    def out_dma(k, b):
        bid = base_blk + k
        f = bid // TB
        tb = bid % TB
        pltpu.async_copy(
            ot_v.at[b], out_hbm.at[f, :, pl.ds(tb * 128, 128)], osems[b])

    def wait_out(b):
        pltpu.make_async_copy(
            ot_v.at[b], out_hbm.at[0, :, pl.ds(0, 128)], osems[b]).wait()

    prep(0, 0)
    fire(0)

    def body(m, carry):
        k0 = m * 2
        k1 = k0 + 1
        prep(k1, 1)
        fire(1)
        drain(0)

        @pl.when(m >= 1)
        def _():
            wait_out(0)

        extract(0)
        out_dma(k0, 0)
        @pl.when(m + 1 < NPAIR)
        def _():
            prep(k0 + 2, 0)
            fire(0)

        drain(1)

        @pl.when(m >= 1)
        def _():
            wait_out(1)

        extract(1)
        out_dma(k1, 1)
        return carry

    lax.fori_loop(0, NPAIR, body, 0)
    wait_out(0)
    wait_out(1)


def _tc_format_body(x_ref, o_ref):
    x = x_ref[:]
    o_ref[:] = x.reshape(DIM, 128, 4).transpose(1, 2, 0).reshape(128, 128)


def _tc_format(tw):
    return pl.pallas_call(
        _tc_format_body,
        out_shape=jax.ShapeDtypeStruct((NUM_EMB // 4, 128), jnp.float32),
        grid=(1954,),
        in_specs=[pl.BlockSpec((DIM, 512), lambda j: (0, j))],
        out_specs=pl.BlockSpec((128, 128), lambda j: (j, 0)),
    )(tw)


def kernel(input, weight):
    idx = input.T.reshape(-1)
    table4 = _tc_format(weight.T)
    out_phys = _embed(idx, table4)
    return jnp.transpose(out_phys, (2, 0, 1))


# speedup vs baseline: 2.2339x; 2.2339x over previous
import functools

import jax
import jax.numpy as jnp
from jax import lax
from jax.experimental import pallas as pl
from jax.experimental.pallas import tpu as pltpu
from jax.experimental.pallas import tpu_sc as plsc

NUM_EMB = 1000000
DIM = 32
BATCH = 16384
FIELDS = 26
B = BATCH * FIELDS
SUPER4 = 1954 * 128

NC = 2
NS = 16
NW = NC * NS
TB = BATCH // 128
NBLK = FIELDS * TB
BLK_PER_W = NBLK // NW
NPAIR = BLK_PER_W // 2
IDX_PER_W = BLK_PER_W * 128

_mesh = plsc.VectorSubcoreMesh(core_axis_name="c", subcore_axis_name="s")

NPANEL = 7812
SLAB = 128
NSLAB = NPANEL * 128 // SLAB
SLAB_PER_W = 246


@functools.partial(
    pl.kernel,
    mesh=_mesh,
    out_type=jax.ShapeDtypeStruct((NUM_EMB // 4, 128), jnp.float32),
    compiler_params=pltpu.CompilerParams(
        use_tc_tiling_on_sc=True, needs_layout_passes=False),
    scratch_types=[
        pltpu.VMEM((2, DIM, SLAB + 1), jnp.float32),
        pltpu.VMEM((2, SLAB // 4, 128), jnp.float32),
        pltpu.SemaphoreType.DMA,
        pltpu.SemaphoreType.DMA,
        pltpu.SemaphoreType.DMA,
        pltpu.SemaphoreType.DMA,
    ],
)
def _format(tw_hbm, tail4_hbm, out_hbm, src_v, dst_v, rsem0, rsem1, wsem0,
            wsem1):
    wid = lax.axis_index("s") * NC + lax.axis_index("c")
    rsems = (rsem0, rsem1)
    wsems = (wsem0, wsem1)

    def slab_of(t):
        return t * NW + wid

    def fire_read(t, b):
        pltpu.async_copy(
            tw_hbm.at[:, pl.ds(slab_of(t) * SLAB, SLAB)],
            src_v.at[b, :, pl.ds(0, SLAB)], rsems[b])

    def drain_read(t, b):
        pltpu.make_async_copy(
            tw_hbm.at[:, pl.ds(slab_of(t) * SLAB, SLAB)],
            src_v.at[b, :, pl.ds(0, SLAB)], rsems[b]).wait()

    def transpose(b):
        rids = []
        cids = []
        for h in range(8):
            pvec = lax.iota(jnp.int32, 16) + h * 16
            rids.append(lax.bitwise_and(pvec, 31))
            cids.append(lax.shift_right_logical(pvec, 5))

        @plsc.parallel_loop(0, SLAB // 4, step=1, unroll=4)
        def _(s):
            for h in range(8):
                val = plsc.load_gather(
                    src_v.at[b], [rids[h], cids[h] + s * 4])
                dst_v[b, s, pl.ds(h * 16, 16)] = val

    def fire_write(t, b):
        pltpu.async_copy(
            dst_v.at[b],
            out_hbm.at[pl.ds(slab_of(t) * (SLAB // 4), SLAB // 4)], wsems[b])

    def wait_write(b):
        pltpu.make_async_copy(
            dst_v.at[b], out_hbm.at[pl.ds(0, SLAB // 4)], wsems[b]).wait()

    @pl.when(slab_of(0) < NSLAB)
    def _():
        fire_read(0, 0)

    def half(t, b, m):
        @pl.when(slab_of(t + 1) < NSLAB)
        def _():
            fire_read(t + 1, 1 - b)

        @pl.when(slab_of(t) < NSLAB)
        def _():
            drain_read(t, b)

            @pl.when(m >= 1)
            def _():
                wait_write(b)

            transpose(b)
            fire_write(t, b)

    def body(m, carry):
        half(m * 2, 0, m)
        half(m * 2 + 1, 1, m)
        return carry

    lax.fori_loop(0, SLAB_PER_W // 2, body, 0)
    wait_write(0)
    wait_write(1)

    @pl.when(wid == 0)
    def _():
        pltpu.sync_copy(tail4_hbm, dst_v.at[0, pl.ds(0, 16)])
        pltpu.sync_copy(
            dst_v.at[0, pl.ds(0, 16)], out_hbm.at[pl.ds(NPANEL * 32, 16)])


@functools.partial(
    pl.kernel,
    mesh=_mesh,
    out_type=jax.ShapeDtypeStruct((FIELDS, DIM, BATCH), jnp.float32),
    compiler_params=pltpu.CompilerParams(
        use_tc_tiling_on_sc=True, needs_layout_passes=False),
    scratch_types=[
        pltpu.VMEM((IDX_PER_W,), jnp.int32),
        pltpu.VMEM((2, 128), jnp.int32),
        pltpu.VMEM((2, 128), jnp.int32),
        pltpu.VMEM((2, 128, 128), jnp.float32),
        pltpu.VMEM((2, DIM, 128), jnp.float32),
        pltpu.SemaphoreType.DMA,
        pltpu.SemaphoreType.DMA,
        pltpu.SemaphoreType.DMA,
        pltpu.SemaphoreType.DMA,
    ],
)
def _embed(idx_hbm, table_hbm, out_hbm, idx_v, sid_v, cb_v, rows_v, ot_v,
           gsem0, gsem1, osem0, osem1):
    wid = lax.axis_index("s") * NC + lax.axis_index("c")
    base_blk = wid * BLK_PER_W
    pltpu.sync_copy(idx_hbm.at[pl.ds(wid * IDX_PER_W, IDX_PER_W)], idx_v)

    gsems = (gsem0, gsem1)
    osems = (osem0, osem1)

    def prep(k, b):
        for g in range(8):
            v = idx_v[pl.ds(k * 128 + g * 16, 16)]
            sid_v[b, pl.ds(g * 16, 16)] = lax.bitwise_or(
                lax.shift_left(lax.shift_right_logical(v, 9), 7),
                lax.bitwise_and(v, 127))
            cb_v[b, pl.ds(g * 16, 16)] = lax.shift_left(
                lax.bitwise_and(lax.shift_right_logical(v, 7), 3), 5)

    def fire(b):
        pltpu.async_copy(table_hbm.at[sid_v.at[b]], rows_v.at[b], gsems[b])

    def drain(b):
        pltpu.make_async_copy(
            table_hbm.at[sid_v.at[b]], rows_v.at[b], gsems[b]).wait()

    def extract(b):
        rids = [lax.iota(jnp.int32, 16) + g * 16 for g in range(8)]
        cbs = [cb_v[b, pl.ds(g * 16, 16)] for g in range(8)]

        @plsc.parallel_loop(0, DIM, step=1, unroll=4)
        def _(d):
            for g in range(8):
                val = plsc.load_gather(rows_v.at[b], [rids[g], cbs[g] + d])
                ot_v[b, d, pl.ds(g * 16, 16)] = val

    def out_dma(k, b):
        bid = base_blk + k
        f = bid // TB
        tb = bid % TB
        pltpu.async_copy(
            ot_v.at[b], out_hbm.at[f, :, pl.ds(tb * 128, 128)], osems[b])

    def wait_out(b):
        pltpu.make_async_copy(
            ot_v.at[b], out_hbm.at[0, :, pl.ds(0, 128)], osems[b]).wait()

    prep(0, 0)
    fire(0)

    def body(m, carry):
        k0 = m * 2
        k1 = k0 + 1
        prep(k1, 1)
        fire(1)
        drain(0)

        @pl.when(m >= 1)
        def _():
            wait_out(0)

        extract(0)
        out_dma(k0, 0)
        @pl.when(m + 1 < NPAIR)
        def _():
            prep(k0 + 2, 0)
            fire(0)

        drain(1)

        @pl.when(m >= 1)
        def _():
            wait_out(1)

        extract(1)
        out_dma(k1, 1)
        return carry

    lax.fori_loop(0, NPAIR, body, 0)
    wait_out(0)
    wait_out(1)


def _tc_format_body(x_ref, o_ref):
    for q in range(4):
        o_ref[:, q * DIM:(q + 1) * DIM] = x_ref[:, q * 128:(q + 1) * 128].T


def _tc_format(tw):
    return pl.pallas_call(
        _tc_format_body,
        out_shape=jax.ShapeDtypeStruct((SUPER4, 128), jnp.float32),
        grid=(1954,),
        in_specs=[pl.BlockSpec((DIM, 512), lambda j: (0, j))],
        out_specs=pl.BlockSpec((128, 128), lambda j: (j, 0)),
    )(tw)


def kernel(input, weight):
    idx = input.T.reshape(-1)
    table4 = _tc_format(weight.T)
    out_phys = _embed(idx, table4)
    return jnp.transpose(out_phys, (2, 0, 1))


# speedup vs baseline: 4.9860x; 2.2320x over previous
import functools

import jax
import jax.numpy as jnp
from jax import lax
from jax.experimental import pallas as pl
from jax.experimental.pallas import tpu as pltpu
from jax.experimental.pallas import tpu_sc as plsc

NUM_EMB = 1000000
DIM = 32
BATCH = 16384
FIELDS = 26
B = BATCH * FIELDS
SUPER4 = 1954 * 128

NC = 2
NS = 16
NW = NC * NS
TB = BATCH // 128
NBLK = FIELDS * TB
BLK_PER_W = NBLK // NW
NPAIR = BLK_PER_W // 2
IDX_PER_W = BLK_PER_W * 128

_mesh = plsc.VectorSubcoreMesh(core_axis_name="c", subcore_axis_name="s")

NPANEL = 7812
SLAB = 128
NSLAB = NPANEL * 128 // SLAB
SLAB_PER_W = 246


@functools.partial(
    pl.kernel,
    mesh=_mesh,
    out_type=jax.ShapeDtypeStruct((NUM_EMB // 4, 128), jnp.float32),
    compiler_params=pltpu.CompilerParams(
        use_tc_tiling_on_sc=True, needs_layout_passes=False),
    scratch_types=[
        pltpu.VMEM((2, DIM, SLAB + 1), jnp.float32),
        pltpu.VMEM((2, SLAB // 4, 128), jnp.float32),
        pltpu.SemaphoreType.DMA,
        pltpu.SemaphoreType.DMA,
        pltpu.SemaphoreType.DMA,
        pltpu.SemaphoreType.DMA,
    ],
)
def _format(tw_hbm, tail4_hbm, out_hbm, src_v, dst_v, rsem0, rsem1, wsem0,
            wsem1):
    wid = lax.axis_index("s") * NC + lax.axis_index("c")
    rsems = (rsem0, rsem1)
    wsems = (wsem0, wsem1)

    def slab_of(t):
        return t * NW + wid

    def fire_read(t, b):
        pltpu.async_copy(
            tw_hbm.at[:, pl.ds(slab_of(t) * SLAB, SLAB)],
            src_v.at[b, :, pl.ds(0, SLAB)], rsems[b])

    def drain_read(t, b):
        pltpu.make_async_copy(
            tw_hbm.at[:, pl.ds(slab_of(t) * SLAB, SLAB)],
            src_v.at[b, :, pl.ds(0, SLAB)], rsems[b]).wait()

    def transpose(b):
        rids = []
        cids = []
        for h in range(8):
            pvec = lax.iota(jnp.int32, 16) + h * 16
            rids.append(lax.bitwise_and(pvec, 31))
            cids.append(lax.shift_right_logical(pvec, 5))

        @plsc.parallel_loop(0, SLAB // 4, step=1, unroll=8)
        def _(s):
            for h in range(8):
                val = plsc.load_gather(
                    src_v.at[b], [rids[h], cids[h] + s * 4])
                dst_v[b, s, pl.ds(h * 16, 16)] = val

    def fire_write(t, b):
        pltpu.async_copy(
            dst_v.at[b],
            out_hbm.at[pl.ds(slab_of(t) * (SLAB // 4), SLAB // 4)], wsems[b])

    def wait_write(b):
        pltpu.make_async_copy(
            dst_v.at[b], out_hbm.at[pl.ds(0, SLAB // 4)], wsems[b]).wait()

    @pl.when(slab_of(0) < NSLAB)
    def _():
        fire_read(0, 0)

    def half(t, b, m):
        @pl.when(slab_of(t + 1) < NSLAB)
        def _():
            fire_read(t + 1, 1 - b)

        @pl.when(slab_of(t) < NSLAB)
        def _():
            drain_read(t, b)

            @pl.when(m >= 1)
            def _():
                wait_write(b)

            transpose(b)
            fire_write(t, b)

    def body(m, carry):
        half(m * 2, 0, m)
        half(m * 2 + 1, 1, m)
        return carry

    lax.fori_loop(0, SLAB_PER_W // 2, body, 0)
    wait_write(0)
    wait_write(1)

    @pl.when(wid == 0)
    def _():
        pltpu.sync_copy(tail4_hbm, dst_v.at[0, pl.ds(0, 16)])
        pltpu.sync_copy(
            dst_v.at[0, pl.ds(0, 16)], out_hbm.at[pl.ds(NPANEL * 32, 16)])


@functools.partial(
    pl.kernel,
    mesh=_mesh,
    out_type=jax.ShapeDtypeStruct((FIELDS, DIM, BATCH), jnp.float32),
    compiler_params=pltpu.CompilerParams(
        use_tc_tiling_on_sc=True, needs_layout_passes=False),
    scratch_types=[
        pltpu.VMEM((IDX_PER_W,), jnp.int32),
        pltpu.VMEM((2, 128), jnp.int32),
        pltpu.VMEM((2, 128), jnp.int32),
        pltpu.VMEM((2, 128, 128), jnp.float32),
        pltpu.VMEM((2, DIM, 128), jnp.float32),
        pltpu.SemaphoreType.DMA,
        pltpu.SemaphoreType.DMA,
        pltpu.SemaphoreType.DMA,
        pltpu.SemaphoreType.DMA,
    ],
)
def _embed(idx_hbm, table_hbm, out_hbm, idx_v, sid_v, cb_v, rows_v, ot_v,
           gsem0, gsem1, osem0, osem1):
    wid = lax.axis_index("s") * NC + lax.axis_index("c")
    base_blk = wid * BLK_PER_W
    pltpu.sync_copy(idx_hbm.at[pl.ds(wid * IDX_PER_W, IDX_PER_W)], idx_v)

    gsems = (gsem0, gsem1)
    osems = (osem0, osem1)

    def prep(k, b):
        for g in range(8):
            v = idx_v[pl.ds(k * 128 + g * 16, 16)]
            sid_v[b, pl.ds(g * 16, 16)] = lax.shift_right_logical(v, 2)
            cb_v[b, pl.ds(g * 16, 16)] = lax.shift_left(
                lax.bitwise_and(v, 3), 5)

    def fire(b):
        pltpu.async_copy(table_hbm.at[sid_v.at[b]], rows_v.at[b], gsems[b])

    def drain(b):
        pltpu.make_async_copy(
            table_hbm.at[sid_v.at[b]], rows_v.at[b], gsems[b]).wait()

    def extract(b):
        rids = [lax.iota(jnp.int32, 16) + g * 16 for g in range(8)]
        cbs = [cb_v[b, pl.ds(g * 16, 16)] for g in range(8)]

        @plsc.parallel_loop(0, DIM, step=1, unroll=4)
        def _(d):
            for g in range(8):
                val = plsc.load_gather(rows_v.at[b], [rids[g], cbs[g] + d])
                ot_v[b, d, pl.ds(g * 16, 16)] = val

    def out_dma(k, b):
        bid = base_blk + k
        f = bid // TB
        tb = bid % TB
        pltpu.async_copy(
            ot_v.at[b], out_hbm.at[f, :, pl.ds(tb * 128, 128)], osems[b])

    def wait_out(b):
        pltpu.make_async_copy(
            ot_v.at[b], out_hbm.at[0, :, pl.ds(0, 128)], osems[b]).wait()

    prep(0, 0)
    fire(0)

    def body(m, carry):
        k0 = m * 2
        k1 = k0 + 1
        prep(k1, 1)
        fire(1)
        drain(0)

        @pl.when(m >= 1)
        def _():
            wait_out(0)

        extract(0)
        out_dma(k0, 0)
        @pl.when(m + 1 < NPAIR)
        def _():
            prep(k0 + 2, 0)
            fire(0)

        drain(1)

        @pl.when(m >= 1)
        def _():
            wait_out(1)

        extract(1)
        out_dma(k1, 1)
        return carry

    lax.fori_loop(0, NPAIR, body, 0)
    wait_out(0)
    wait_out(1)


def kernel(input, weight):
    idx = input.T.reshape(-1)
    tail4 = weight[NPANEL * 128:].reshape(16, 128)
    table4 = _format(weight.T, tail4)
    out_phys = _embed(idx, table4)
    return jnp.transpose(out_phys, (2, 0, 1))


# speedup vs baseline: 4.9878x; 1.0004x over previous
import functools

import jax
import jax.numpy as jnp
from jax import lax
from jax.experimental import pallas as pl
from jax.experimental.pallas import tpu as pltpu
from jax.experimental.pallas import tpu_sc as plsc

NUM_EMB = 1000000
DIM = 32
BATCH = 16384
FIELDS = 26
B = BATCH * FIELDS
SUPER4 = 1954 * 128

NC = 2
NS = 16
NW = NC * NS
TB = BATCH // 128
NBLK = FIELDS * TB
BLK_PER_W = NBLK // NW
NPAIR = BLK_PER_W // 2
IDX_PER_W = BLK_PER_W * 128

_mesh = plsc.VectorSubcoreMesh(core_axis_name="c", subcore_axis_name="s")

NPANEL = 7812
SLAB = 128
NSLAB = NPANEL * 128 // SLAB
SLAB_PER_W = 246


@functools.partial(
    pl.kernel,
    mesh=_mesh,
    out_type=jax.ShapeDtypeStruct((NUM_EMB // 4, 128), jnp.float32),
    compiler_params=pltpu.CompilerParams(
        use_tc_tiling_on_sc=True, needs_layout_passes=False),
    scratch_types=[
        pltpu.VMEM((2, DIM, SLAB + 1), jnp.float32),
        pltpu.VMEM((2, SLAB // 4, 128), jnp.float32),
        pltpu.SemaphoreType.DMA,
        pltpu.SemaphoreType.DMA,
        pltpu.SemaphoreType.DMA,
        pltpu.SemaphoreType.DMA,
    ],
)
def _format(tw_hbm, tail4_hbm, out_hbm, src_v, dst_v, rsem0, rsem1, wsem0,
            wsem1):
    wid = lax.axis_index("s") * NC + lax.axis_index("c")
    rsems = (rsem0, rsem1)
    wsems = (wsem0, wsem1)

    def slab_of(t):
        return t * NW + wid

    def fire_read(t, b):
        pltpu.async_copy(
            tw_hbm.at[:, pl.ds(slab_of(t) * SLAB, SLAB)],
            src_v.at[b, :, pl.ds(0, SLAB)], rsems[b])

    def drain_read(t, b):
        pltpu.make_async_copy(
            tw_hbm.at[:, pl.ds(slab_of(t) * SLAB, SLAB)],
            src_v.at[b, :, pl.ds(0, SLAB)], rsems[b]).wait()

    def transpose(b):
        rids = []
        cids = []
        for h in range(8):
            pvec = lax.iota(jnp.int32, 16) + h * 16
            rids.append(lax.bitwise_and(pvec, 31))
            cids.append(lax.shift_right_logical(pvec, 5))

        @plsc.parallel_loop(0, SLAB // 4, step=1, unroll=8)
        def _(s):
            for h in range(8):
                val = plsc.load_gather(
                    src_v.at[b], [rids[h], cids[h] + s * 4])
                dst_v[b, s, pl.ds(h * 16, 16)] = val

    def fire_write(t, b):
        pltpu.async_copy(
            dst_v.at[b],
            out_hbm.at[pl.ds(slab_of(t) * (SLAB // 4), SLAB // 4)], wsems[b])

    def wait_write(b):
        pltpu.make_async_copy(
            dst_v.at[b], out_hbm.at[pl.ds(0, SLAB // 4)], wsems[b]).wait()

    @pl.when(slab_of(0) < NSLAB)
    def _():
        fire_read(0, 0)

    def half(t, b, m):
        @pl.when(slab_of(t + 1) < NSLAB)
        def _():
            fire_read(t + 1, 1 - b)

        @pl.when(slab_of(t) < NSLAB)
        def _():
            drain_read(t, b)

            @pl.when(m >= 1)
            def _():
                wait_write(b)

            transpose(b)
            fire_write(t, b)

    def body(m, carry):
        half(m * 2, 0, m)
        half(m * 2 + 1, 1, m)
        return carry

    lax.fori_loop(0, SLAB_PER_W // 2, body, 0)
    wait_write(0)
    wait_write(1)

    @pl.when(wid == 0)
    def _():
        pltpu.sync_copy(tail4_hbm, dst_v.at[0, pl.ds(0, 16)])
        pltpu.sync_copy(
            dst_v.at[0, pl.ds(0, 16)], out_hbm.at[pl.ds(NPANEL * 32, 16)])


@functools.partial(
    pl.kernel,
    mesh=_mesh,
    out_type=jax.ShapeDtypeStruct((FIELDS, DIM, BATCH), jnp.float32),
    compiler_params=pltpu.CompilerParams(
        use_tc_tiling_on_sc=True, needs_layout_passes=False),
    scratch_types=[
        pltpu.VMEM((IDX_PER_W,), jnp.int32),
        pltpu.VMEM((2, 128), jnp.int32),
        pltpu.VMEM((2, 128), jnp.int32),
        pltpu.VMEM((2, 128, 128), jnp.float32),
        pltpu.VMEM((2, DIM, 128), jnp.float32),
        pltpu.SemaphoreType.DMA,
        pltpu.SemaphoreType.DMA,
        pltpu.SemaphoreType.DMA,
        pltpu.SemaphoreType.DMA,
    ],
)
def _embed(idx_hbm, table_hbm, out_hbm, idx_v, sid_v, cb_v, rows_v, ot_v,
           gsem0, gsem1, osem0, osem1):
    wid = lax.axis_index("s") * NC + lax.axis_index("c")
    base_blk = wid * BLK_PER_W
    pltpu.sync_copy(idx_hbm.at[pl.ds(wid * IDX_PER_W, IDX_PER_W)], idx_v)

    gsems = (gsem0, gsem1)
    osems = (osem0, osem1)

    def prep(k, b):
        for g in range(8):
            v = idx_v[pl.ds(k * 128 + g * 16, 16)]
            sid_v[b, pl.ds(g * 16, 16)] = lax.shift_right_logical(v, 2)
            cb_v[b, pl.ds(g * 16, 16)] = lax.shift_left(
                lax.bitwise_and(v, 3), 5)

    def fire(b):
        pltpu.async_copy(table_hbm.at[sid_v.at[b]], rows_v.at[b], gsems[b])

    def drain(b):
        pltpu.make_async_copy(
            table_hbm.at[sid_v.at[b]], rows_v.at[b], gsems[b]).wait()

    def extract(b):
        rids = [lax.iota(jnp.int32, 16) + g * 16 for g in range(8)]
        cbs = [cb_v[b, pl.ds(g * 16, 16)] for g in range(8)]

        @plsc.parallel_loop(0, DIM, step=1, unroll=8)
        def _(d):
            for g in range(8):
                val = plsc.load_gather(rows_v.at[b], [rids[g], cbs[g] + d])
                ot_v[b, d, pl.ds(g * 16, 16)] = val

    def out_dma(k, b):
        bid = base_blk + k
        f = bid // TB
        tb = bid % TB
        pltpu.async_copy(
            ot_v.at[b], out_hbm.at[f, :, pl.ds(tb * 128, 128)], osems[b])

    def wait_out(b):
        pltpu.make_async_copy(
            ot_v.at[b], out_hbm.at[0, :, pl.ds(0, 128)], osems[b]).wait()

    prep(0, 0)
    fire(0)

    def body(m, carry):
        k0 = m * 2
        k1 = k0 + 1
        prep(k1, 1)
        fire(1)
        drain(0)

        @pl.when(m >= 1)
        def _():
            wait_out(0)

        extract(0)
        out_dma(k0, 0)
        @pl.when(m + 1 < NPAIR)
        def _():
            prep(k0 + 2, 0)
            fire(0)

        drain(1)

        @pl.when(m >= 1)
        def _():
            wait_out(1)

        extract(1)
        out_dma(k1, 1)
        return carry

    lax.fori_loop(0, NPAIR, body, 0)
    wait_out(0)
    wait_out(1)


def kernel(input, weight):
    idx = input.T.reshape(-1)
    tail4 = weight[NPANEL * 128:].reshape(16, 128)
    table4 = _format(weight.T, tail4)
    out_phys = _embed(idx, table4)
    return jnp.transpose(out_phys, (2, 0, 1))


# speedup vs baseline: 4.9933x; 1.0011x over previous
import functools

import jax
import jax.numpy as jnp
from jax import lax
from jax.experimental import pallas as pl
from jax.experimental.pallas import tpu as pltpu
from jax.experimental.pallas import tpu_sc as plsc

NUM_EMB = 1000000
DIM = 32
BATCH = 16384
FIELDS = 26
B = BATCH * FIELDS
SUPER4 = 1954 * 128

NC = 2
NS = 16
NW = NC * NS
TB = BATCH // 128
NBLK = FIELDS * TB
BLK_PER_W = NBLK // NW
NPAIR = BLK_PER_W // 2
IDX_PER_W = BLK_PER_W * 128

_mesh = plsc.VectorSubcoreMesh(core_axis_name="c", subcore_axis_name="s")

NPANEL = 7812
SLAB = 128
NSLAB = NPANEL * 128 // SLAB
SLAB_PER_W = 248


@functools.partial(
    pl.kernel,
    mesh=_mesh,
    out_type=jax.ShapeDtypeStruct((NUM_EMB // 4, 128), jnp.float32),
    compiler_params=pltpu.CompilerParams(
        use_tc_tiling_on_sc=True, needs_layout_passes=False),
    scratch_types=[
        pltpu.VMEM((4, DIM, SLAB + 1), jnp.float32),
        pltpu.VMEM((4, SLAB // 4, 128), jnp.float32),
        pltpu.SemaphoreType.DMA,
        pltpu.SemaphoreType.DMA,
        pltpu.SemaphoreType.DMA,
        pltpu.SemaphoreType.DMA,
        pltpu.SemaphoreType.DMA,
        pltpu.SemaphoreType.DMA,
        pltpu.SemaphoreType.DMA,
        pltpu.SemaphoreType.DMA,
    ],
)
def _format(tw_hbm, tail4_hbm, out_hbm, src_v, dst_v, rsem0, rsem1, rsem2,
            rsem3, wsem0, wsem1, wsem2, wsem3):
    wid = lax.axis_index("s") * NC + lax.axis_index("c")
    rsems = (rsem0, rsem1, rsem2, rsem3)
    wsems = (wsem0, wsem1, wsem2, wsem3)

    def slab_of(t):
        return t * NW + wid

    def fire_read(t, b):
        pltpu.async_copy(
            tw_hbm.at[:, pl.ds(slab_of(t) * SLAB, SLAB)],
            src_v.at[b, :, pl.ds(0, SLAB)], rsems[b])

    def drain_read(t, b):
        pltpu.make_async_copy(
            tw_hbm.at[:, pl.ds(slab_of(t) * SLAB, SLAB)],
            src_v.at[b, :, pl.ds(0, SLAB)], rsems[b]).wait()

    def transpose(b):
        rids = []
        cids = []
        for h in range(8):
            pvec = lax.iota(jnp.int32, 16) + h * 16
            rids.append(lax.bitwise_and(pvec, 31))
            cids.append(lax.shift_right_logical(pvec, 5))

        @plsc.parallel_loop(0, SLAB // 4, step=1, unroll=8)
        def _(s):
            for h in range(8):
                val = plsc.load_gather(
                    src_v.at[b], [rids[h], cids[h] + s * 4])
                dst_v[b, s, pl.ds(h * 16, 16)] = val

    def fire_write(t, b):
        pltpu.async_copy(
            dst_v.at[b],
            out_hbm.at[pl.ds(slab_of(t) * (SLAB // 4), SLAB // 4)], wsems[b])

    def wait_write(b):
        pltpu.make_async_copy(
            dst_v.at[b], out_hbm.at[pl.ds(0, SLAB // 4)], wsems[b]).wait()

    for t0 in range(3):
        @pl.when(slab_of(t0) < NSLAB)
        def _():
            fire_read(t0, t0)

    def half(t, b, m):
        @pl.when(slab_of(t + 3) < NSLAB)
        def _():
            fire_read(t + 3, (b + 3) % 4)

        @pl.when(slab_of(t) < NSLAB)
        def _():
            drain_read(t, b)

            @pl.when(m >= 1)
            def _():
                wait_write(b)

            transpose(b)
            fire_write(t, b)

    def body(m, carry):
        for i in range(4):
            half(m * 4 + i, i, m)
        return carry

    lax.fori_loop(0, SLAB_PER_W // 4, body, 0)
    for b in range(4):
        wait_write(b)

    @pl.when(wid == 0)
    def _():
        pltpu.sync_copy(tail4_hbm, dst_v.at[0, pl.ds(0, 16)])
        pltpu.sync_copy(
            dst_v.at[0, pl.ds(0, 16)], out_hbm.at[pl.ds(NPANEL * 32, 16)])


@functools.partial(
    pl.kernel,
    mesh=_mesh,
    out_type=jax.ShapeDtypeStruct((FIELDS, DIM, BATCH), jnp.float32),
    compiler_params=pltpu.CompilerParams(
        use_tc_tiling_on_sc=True, needs_layout_passes=False),
    scratch_types=[
        pltpu.VMEM((IDX_PER_W,), jnp.int32),
        pltpu.VMEM((2, 128), jnp.int32),
        pltpu.VMEM((2, 128), jnp.int32),
        pltpu.VMEM((2, 128, 128), jnp.float32),
        pltpu.VMEM((2, DIM, 128), jnp.float32),
        pltpu.SemaphoreType.DMA,
        pltpu.SemaphoreType.DMA,
        pltpu.SemaphoreType.DMA,
        pltpu.SemaphoreType.DMA,
    ],
)
def _embed(idx_hbm, table_hbm, out_hbm, idx_v, sid_v, cb_v, rows_v, ot_v,
           gsem0, gsem1, osem0, osem1):
    wid = lax.axis_index("s") * NC + lax.axis_index("c")
    base_blk = wid * BLK_PER_W
    pltpu.sync_copy(idx_hbm.at[pl.ds(wid * IDX_PER_W, IDX_PER_W)], idx_v)

    gsems = (gsem0, gsem1)
    osems = (osem0, osem1)

    def prep(k, b):
        for g in range(8):
            v = idx_v[pl.ds(k * 128 + g * 16, 16)]
            sid_v[b, pl.ds(g * 16, 16)] = lax.shift_right_logical(v, 2)
            cb_v[b, pl.ds(g * 16, 16)] = lax.shift_left(
                lax.bitwise_and(v, 3), 5)

    def fire(b):
        pltpu.async_copy(table_hbm.at[sid_v.at[b]], rows_v.at[b], gsems[b])

    def drain(b):
        pltpu.make_async_copy(
            table_hbm.at[sid_v.at[b]], rows_v.at[b], gsems[b]).wait()

    def extract(b):
        rids = [lax.iota(jnp.int32, 16) + g * 16 for g in range(8)]
        cbs = [cb_v[b, pl.ds(g * 16, 16)] for g in range(8)]

        @plsc.parallel_loop(0, DIM, step=1, unroll=8)
        def _(d):
            for g in range(8):
                val = plsc.load_gather(rows_v.at[b], [rids[g], cbs[g] + d])
                ot_v[b, d, pl.ds(g * 16, 16)] = val

    def out_dma(k, b):
        bid = base_blk + k
        f = bid // TB
        tb = bid % TB
        pltpu.async_copy(
            ot_v.at[b], out_hbm.at[f, :, pl.ds(tb * 128, 128)], osems[b])

    def wait_out(b):
        pltpu.make_async_copy(
            ot_v.at[b], out_hbm.at[0, :, pl.ds(0, 128)], osems[b]).wait()

    prep(0, 0)
    fire(0)

    def body(m, carry):
        k0 = m * 2
        k1 = k0 + 1
        prep(k1, 1)
        fire(1)
        drain(0)

        @pl.when(m >= 1)
        def _():
            wait_out(0)

        extract(0)
        out_dma(k0, 0)
        @pl.when(m + 1 < NPAIR)
        def _():
            prep(k0 + 2, 0)
            fire(0)

        drain(1)

        @pl.when(m >= 1)
        def _():
            wait_out(1)

        extract(1)
        out_dma(k1, 1)
        return carry

    lax.fori_loop(0, NPAIR, body, 0)
    wait_out(0)
    wait_out(1)


def kernel(input, weight):
    idx = input.T.reshape(-1)
    tail4 = weight[NPANEL * 128:].reshape(16, 128)
    table4 = _format(weight.T, tail4)
    out_phys = _embed(idx, table4)
    return jnp.transpose(out_phys, (2, 0, 1))
